# Initial kernel scaffold; baseline (speedup 1.0000x reference)
import jax
import jax.numpy as jnp
from jax.experimental import pallas as pl


def kernel(logits, labels):
    raise NotImplementedError("write your pallas kernel here")


# speedup vs baseline: 38.2822x; 38.2822x over previous
import functools
import numpy as np
import jax
import jax.numpy as jnp
from jax.experimental import pallas as pl
from jax.experimental.pallas import tpu as pltpu

_THRESH = float(-np.log(0.7))
_NMIN_FRAC = 0.1
_IGNORE = 255

_BH = 64


def _ce_loss_tile(z_ref, lab_ref):
    C = z_ref.shape[1]
    lab = lab_ref[0]
    m = z_ref[0, 0]
    for c in range(1, C):
        m = jnp.maximum(m, z_ref[0, c])
    s = jnp.zeros_like(m)
    picked = jnp.zeros_like(m)
    for c in range(C):
        zc = z_ref[0, c]
        s = s + jnp.exp(zc - m)
        picked = picked + jnp.where(lab == c, zc, 0.0)
    loss = m + jnp.log(s) - picked
    return jnp.where(lab == _IGNORE, 0.0, loss)


def _ce_stats_body(z_ref, lab_ref, out_ref):
    loss = _ce_loss_tile(z_ref, lab_ref)
    mask = loss > _THRESH
    c = jnp.sum(mask.astype(jnp.float32))
    sm = jnp.sum(jnp.where(mask, loss, 0.0))
    first = (pl.program_id(0) == 0) & (pl.program_id(1) == 0)

    @pl.when(first)
    def _():
        out_ref[0] = 0.0
        out_ref[1] = 0.0

    out_ref[0] += c
    out_ref[1] += sm


def _ce_loss_body(z_ref, lab_ref, out_ref):
    out_ref[0] = _ce_loss_tile(z_ref, lab_ref)


def _select_body(loss_ref, kp_ref, out_ref):
    x = loss_ref[...]
    kp = kp_ref[0]
    in_s = x <= _THRESH

    def it(_, carry):
        lo, hi = carry
        mid = 0.5 * (lo + hi)
        f = jnp.sum((in_s & (x > mid)).astype(jnp.float32))
        gt = f > kp
        return jnp.where(gt, mid, lo), jnp.where(gt, hi, mid)

    lo, hi = jax.lax.fori_loop(
        0, 50, it, (jnp.float32(-1.0), jnp.float32(_THRESH))
    )
    sel = in_s & (x > hi)
    fhi = jnp.sum(sel.astype(jnp.float32))
    shi = jnp.sum(jnp.where(sel, x, 0.0))
    out_ref[0] = shi + (kp - fhi) * hi


def _run_ce_stats(logits, labels):
    B, C, H, W = logits.shape
    return pl.pallas_call(
        _ce_stats_body,
        grid=(B, H // _BH),
        in_specs=[
            pl.BlockSpec((1, C, _BH, W), lambda b, h: (b, 0, h, 0)),
            pl.BlockSpec((1, _BH, W), lambda b, h: (b, h, 0)),
        ],
        out_specs=pl.BlockSpec(memory_space=pltpu.SMEM),
        out_shape=jax.ShapeDtypeStruct((2,), jnp.float32),
        compiler_params=pltpu.CompilerParams(
            dimension_semantics=("arbitrary", "arbitrary")
        ),
    )(logits, labels)


def _topk_mean(logits, labels, cnt, ssum, n_min):
    B, C, H, W = logits.shape
    loss = pl.pallas_call(
        _ce_loss_body,
        grid=(B, H // _BH),
        in_specs=[
            pl.BlockSpec((1, C, _BH, W), lambda b, h: (b, 0, h, 0)),
            pl.BlockSpec((1, _BH, W), lambda b, h: (b, h, 0)),
        ],
        out_specs=pl.BlockSpec((1, _BH, W), lambda b, h: (b, h, 0)),
        out_shape=jax.ShapeDtypeStruct((B, H, W), jnp.float32),
        compiler_params=pltpu.CompilerParams(
            dimension_semantics=("arbitrary", "arbitrary")
        ),
    )(logits, labels)
    loss2d = loss.reshape(B * H, W)
    kp = (jnp.float32(n_min) - cnt).reshape(1)
    rest = pl.pallas_call(
        _select_body,
        in_specs=[
            pl.BlockSpec(loss2d.shape, lambda: (0, 0)),
            pl.BlockSpec(memory_space=pltpu.SMEM),
        ],
        out_specs=pl.BlockSpec(memory_space=pltpu.SMEM),
        out_shape=jax.ShapeDtypeStruct((1,), jnp.float32),
    )(loss2d, kp)
    return (ssum + rest[0]) / jnp.float32(n_min)


def kernel(logits, labels):
    B, C, H, W = logits.shape
    labels = labels.astype(jnp.int32)
    n = B * H * W
    n_min = int(_NMIN_FRAC * n)
    stats = _run_ce_stats(logits, labels)
    cnt, ssum = stats[0], stats[1]
    mean_thresh = ssum / jnp.maximum(cnt, 1.0)
    return jax.lax.cond(
        cnt > jnp.float32(n_min),
        lambda: mean_thresh,
        lambda: _topk_mean(logits, labels, cnt, ssum, n_min),
    )


# speedup vs baseline: 38.5469x; 1.0069x over previous
import functools
import numpy as np
import jax
import jax.numpy as jnp
from jax.experimental import pallas as pl
from jax.experimental.pallas import tpu as pltpu

_THRESH = float(-np.log(0.7))
_NMIN_FRAC = 0.1
_IGNORE = 255

_BH = 64


def _ce_loss_tile(z_ref, lab_ref):
    C = z_ref.shape[1]
    lab = lab_ref[0]
    m = z_ref[0, 0]
    for c in range(1, C):
        m = jnp.maximum(m, z_ref[0, c])
    s = jnp.zeros_like(m)
    picked = jnp.zeros_like(m)
    for c in range(C):
        zc = z_ref[0, c]
        s = s + jnp.exp(zc - m)
        picked = jnp.where(lab == c, zc, picked)
    loss = m + jnp.log(s) - picked
    return jnp.where(lab == _IGNORE, 0.0, loss)


def _ce_stats_body(z_ref, lab_ref, out_ref):
    loss = _ce_loss_tile(z_ref, lab_ref)
    mask = loss > _THRESH
    c = jnp.sum(mask.astype(jnp.float32))
    sm = jnp.sum(jnp.where(mask, loss, 0.0))
    first = (pl.program_id(0) == 0) & (pl.program_id(1) == 0)

    @pl.when(first)
    def _():
        out_ref[0] = 0.0
        out_ref[1] = 0.0

    out_ref[0] += c
    out_ref[1] += sm


def _ce_loss_body(z_ref, lab_ref, out_ref):
    out_ref[0] = _ce_loss_tile(z_ref, lab_ref)


def _select_body(loss_ref, kp_ref, out_ref):
    x = loss_ref[...]
    kp = kp_ref[0]
    in_s = x <= _THRESH

    def it(_, carry):
        lo, hi = carry
        mid = 0.5 * (lo + hi)
        f = jnp.sum((in_s & (x > mid)).astype(jnp.float32))
        gt = f > kp
        return jnp.where(gt, mid, lo), jnp.where(gt, hi, mid)

    lo, hi = jax.lax.fori_loop(
        0, 50, it, (jnp.float32(-1.0), jnp.float32(_THRESH))
    )
    sel = in_s & (x > hi)
    fhi = jnp.sum(sel.astype(jnp.float32))
    shi = jnp.sum(jnp.where(sel, x, 0.0))
    out_ref[0] = shi + (kp - fhi) * hi


def _run_ce_stats(logits, labels):
    B, C, H, W = logits.shape
    return pl.pallas_call(
        _ce_stats_body,
        grid=(B, H // _BH),
        in_specs=[
            pl.BlockSpec((1, C, _BH, W), lambda b, h: (b, 0, h, 0)),
            pl.BlockSpec((1, _BH, W), lambda b, h: (b, h, 0)),
        ],
        out_specs=pl.BlockSpec(memory_space=pltpu.SMEM),
        out_shape=jax.ShapeDtypeStruct((2,), jnp.float32),
        compiler_params=pltpu.CompilerParams(
            dimension_semantics=("arbitrary", "arbitrary")
        ),
    )(logits, labels)


def _topk_mean(logits, labels, cnt, ssum, n_min):
    B, C, H, W = logits.shape
    loss = pl.pallas_call(
        _ce_loss_body,
        grid=(B, H // _BH),
        in_specs=[
            pl.BlockSpec((1, C, _BH, W), lambda b, h: (b, 0, h, 0)),
            pl.BlockSpec((1, _BH, W), lambda b, h: (b, h, 0)),
        ],
        out_specs=pl.BlockSpec((1, _BH, W), lambda b, h: (b, h, 0)),
        out_shape=jax.ShapeDtypeStruct((B, H, W), jnp.float32),
        compiler_params=pltpu.CompilerParams(
            dimension_semantics=("arbitrary", "arbitrary")
        ),
    )(logits, labels)
    loss2d = loss.reshape(B * H, W)
    kp = (jnp.float32(n_min) - cnt).reshape(1)
    rest = pl.pallas_call(
        _select_body,
        in_specs=[
            pl.BlockSpec(loss2d.shape, lambda: (0, 0)),
            pl.BlockSpec(memory_space=pltpu.SMEM),
        ],
        out_specs=pl.BlockSpec(memory_space=pltpu.SMEM),
        out_shape=jax.ShapeDtypeStruct((1,), jnp.float32),
    )(loss2d, kp)
    return (ssum + rest[0]) / jnp.float32(n_min)


def kernel(logits, labels):
    B, C, H, W = logits.shape
    labels = labels.astype(jnp.int32)
    n = B * H * W
    n_min = int(_NMIN_FRAC * n)
    stats = _run_ce_stats(logits, labels)
    cnt, ssum = stats[0], stats[1]
    mean_thresh = ssum / jnp.maximum(cnt, 1.0)
    return jax.lax.cond(
        cnt > jnp.float32(n_min),
        lambda: mean_thresh,
        lambda: _topk_mean(logits, labels, cnt, ssum, n_min),
    )
